# Initial kernel scaffold; baseline (speedup 1.0000x reference)
import jax
import jax.numpy as jnp
from jax.experimental import pallas as pl


def kernel(x, edge_index, edge_attr, node2graph, W_lin, b_lin, We1, be1, We2, be2, W_ih, W_hh, b_ih, b_hh, Wl_ih, Wl_hh, bl_ih, bl_hh):
    raise NotImplementedError("write your pallas kernel here")


# speedup vs baseline: 5.6629x; 5.6629x over previous
import functools

import jax
import jax.numpy as jnp
from jax import lax
from jax.experimental import pallas as pl
from jax.experimental.pallas import tpu as pltpu
from jax.experimental.pallas import tpu_sc as plsc

N = 10000
E = 160000
D_IN = 128
H = 16
G = 64
S2S_STEPS = 3

NC = 2
NS = 16
NW = NC * NS
EPW = E // NW
CH = 100
NCH = EPW // CH
STRIPE = N // NS

@functools.cache
def _sc_kernels():
    mesh = plsc.VectorSubcoreMesh(core_axis_name="c", subcore_axis_name="s")

    @functools.partial(
        pl.kernel,
        mesh=mesh,
        out_type=jax.ShapeDtypeStruct((NW, NCH, CH, H), jnp.float32),
        scratch_types=[
            pltpu.VMEM((NCH, CH), jnp.int32),
            pltpu.VMEM((NCH, CH, H), jnp.float32),
            pltpu.SemaphoreType.DMA,
        ],
        compiler_params=pltpu.CompilerParams(use_tc_tiling_on_sc=False),
    )
    def sc_gather(table_hbm, idx_hbm, out_hbm, idx_v, rows_v, sem):
        wid = lax.axis_index("s") * NC + lax.axis_index("c")
        pltpu.sync_copy(idx_hbm.at[wid], idx_v)

        def body(j, carry):
            pltpu.async_copy(table_hbm.at[idx_v.at[j]], rows_v.at[j], sem).wait()
            return carry

        lax.fori_loop(0, NCH, body, 0)
        pltpu.sync_copy(rows_v, out_hbm.at[wid])

    @functools.partial(
        pl.kernel,
        mesh=mesh,
        out_type=jax.ShapeDtypeStruct((NC * N, H), jnp.float32),
        scratch_types=[
            pltpu.VMEM((NCH, CH), jnp.int32),
            pltpu.VMEM((NCH, CH, H), jnp.float32),
            pltpu.VMEM((STRIPE, H), jnp.float32),
            pltpu.VMEM_SHARED((N, H), jnp.float32),
            pltpu.SemaphoreType.DMA,
        ],
        compiler_params=pltpu.CompilerParams(use_tc_tiling_on_sc=False),
    )
    def sc_scatter(msg_hbm, dst_hbm, zeros_hbm, out_hbm,
                   dst_v, msg_v, stripe_v, acc_sh, sem):
        cid = lax.axis_index("c")
        sid = lax.axis_index("s")
        wid = sid * NC + cid
        pltpu.sync_copy(zeros_hbm, stripe_v)
        pltpu.sync_copy(stripe_v, acc_sh.at[pl.ds(sid * STRIPE, STRIPE)])
        plsc.subcore_barrier()
        pltpu.sync_copy(dst_hbm.at[wid], dst_v)
        pltpu.sync_copy(msg_hbm.at[wid], msg_v)

        def body(j, carry):
            pltpu.sync_copy(msg_v.at[j], acc_sh.at[dst_v.at[j]], add=True)
            return carry

        lax.fori_loop(0, NCH, body, 0)
        plsc.subcore_barrier()
        pltpu.sync_copy(acc_sh.at[pl.ds(sid * STRIPE, STRIPE)], stripe_v)
        pltpu.sync_copy(stripe_v,
                        out_hbm.at[pl.ds(cid * N + sid * STRIPE, STRIPE)])

    return sc_gather, sc_scatter


def _lin_body(x_ref, w_ref, b_ref, o_ref):
    o_ref[...] = (
        jnp.dot(x_ref[...], w_ref[...], preferred_element_type=jnp.float32)
        + b_ref[...]
    )


def _edge_body(ea_ref, ls_ref, we1_ref, be1_ref, rk_ref, rj_ref, m2_ref,
               b2t_ref, o_ref):
    f32 = jnp.float32
    ea = ea_ref[...]
    ls = ls_ref[...]
    hmid = jnp.maximum(
        jnp.dot(ea, we1_ref[...], preferred_element_type=f32) + be1_ref[...],
        0.0,
    )
    hr = jnp.dot(hmid, rk_ref[...], preferred_element_type=f32)
    lt = jnp.dot(ls, rj_ref[...], preferred_element_type=f32)
    o_ref[...] = (
        jnp.dot(hr * lt, m2_ref[...], preferred_element_type=f32)
        + jnp.dot(ls, b2t_ref[...], preferred_element_type=f32)
    )


def _tail_body(u2_ref, li_ref, n2g_ref,
               wir_ref, wiz_ref, win_ref, whr_ref, whz_ref, whn_ref,
               br_ref, bz_ref, bin_ref, bhn_ref,
               wlii_ref, wlif_ref, wlig_ref, wlio_ref,
               wlhi_ref, wlhf_ref, wlhg_ref, wlho_ref,
               bli_ref, blf_ref, blg_ref, blo_ref,
               gf_ref, nf_ref):
    f32 = jnp.float32
    dot = lambda a, b: jnp.dot(a, b, preferred_element_type=f32)
    upd = u2_ref[0:N, :] + u2_ref[N:2 * N, :]
    mp = jnp.maximum(upd, 0.0)
    hx = li_ref[...]
    r = jax.nn.sigmoid(dot(mp, wir_ref[...]) + dot(hx, whr_ref[...]) + br_ref[...])
    z = jax.nn.sigmoid(dot(mp, wiz_ref[...]) + dot(hx, whz_ref[...]) + bz_ref[...])
    gn = jnp.tanh(dot(mp, win_ref[...]) + bin_ref[...]
                  + r * (dot(hx, whn_ref[...]) + bhn_ref[...]))
    hidden = (1.0 - z) * gn + z * hx
    nf_ref[...] = hidden
    n2g = n2g_ref[...]
    gids = lax.broadcasted_iota(jnp.int32, (N, G), 1)
    onehot_b = n2g == gids
    onehot_f = onehot_b.astype(f32)
    qstar = jnp.zeros((G, 2 * H), f32)
    h_l = jnp.zeros((G, H), f32)
    c_l = jnp.zeros((G, H), f32)
    for _ in range(S2S_STEPS):
        ig = jax.nn.sigmoid(dot(qstar, wlii_ref[...]) + dot(h_l, wlhi_ref[...]) + bli_ref[...])
        fg = jax.nn.sigmoid(dot(qstar, wlif_ref[...]) + dot(h_l, wlhf_ref[...]) + blf_ref[...])
        gg = jnp.tanh(dot(qstar, wlig_ref[...]) + dot(h_l, wlhg_ref[...]) + blg_ref[...])
        og = jax.nn.sigmoid(dot(qstar, wlio_ref[...]) + dot(h_l, wlho_ref[...]) + blo_ref[...])
        c_l = fg * c_l + ig * gg
        h_l = og * jnp.tanh(c_l)
        q = h_l
        qn = dot(onehot_f, q)
        e = jnp.sum(hidden * qn, axis=1, keepdims=True)
        em = jnp.where(onehot_b, e, -jnp.inf)
        m = jnp.max(em, axis=0, keepdims=True)
        m = jnp.where(m > -jnp.inf, m, 0.0)
        a = jnp.where(onehot_b, jnp.exp(e - m), 0.0)
        s = jnp.sum(a, axis=0, keepdims=True)
        w = a / (s + 1e-12)
        rr = lax.dot_general(w, hidden, (((0,), (0,)), ((), ())),
                             preferred_element_type=f32)
        qstar = jnp.concatenate([q, rr], axis=1)
    gf_ref[...] = qstar


def _linear(x, W, b):
    blk = 2000
    return pl.pallas_call(
        _lin_body,
        grid=(N // blk,),
        in_specs=[
            pl.BlockSpec((blk, D_IN), lambda i: (i, 0)),
            pl.BlockSpec((D_IN, H), lambda i: (0, 0)),
            pl.BlockSpec((1, H), lambda i: (0, 0)),
        ],
        out_specs=pl.BlockSpec((blk, H), lambda i: (i, 0)),
        out_shape=jax.ShapeDtypeStruct((N, H), jnp.float32),
    )(x, W, b.reshape(1, H))


def _edge_messages(edge_attr, ls, We1, be1, Rk, Rj, M2, B2T):
    blk = 4000
    full = lambda shape: pl.BlockSpec(shape, lambda i: tuple(0 for _ in shape))
    return pl.pallas_call(
        _edge_body,
        grid=(E // blk,),
        in_specs=[
            pl.BlockSpec((blk, H), lambda i: (i, 0)),
            pl.BlockSpec((blk, H), lambda i: (i, 0)),
            full((H, H)),
            full((1, H)),
            full((H, H * H)),
            full((H, H * H)),
            full((H * H, H)),
            full((H, H)),
        ],
        out_specs=pl.BlockSpec((blk, H), lambda i: (i, 0)),
        out_shape=jax.ShapeDtypeStruct((E, H), jnp.float32),
    )(edge_attr, ls, We1, be1.reshape(1, H), Rk, Rj, M2, B2T)


def _tail(u2, layer_input, n2g, gru_w, lstm_w):
    return pl.pallas_call(
        _tail_body,
        out_shape=(
            jax.ShapeDtypeStruct((G, 2 * H), jnp.float32),
            jax.ShapeDtypeStruct((N, H), jnp.float32),
        ),
    )(u2, layer_input, n2g, *gru_w, *lstm_w)


def kernel(x, edge_index, edge_attr, node2graph, W_lin, b_lin, We1, be1, We2,
           be2, W_ih, W_hh, b_ih, b_hh, Wl_ih, Wl_hh, bl_ih, bl_hh):
    f32 = jnp.float32
    src = edge_index[0].reshape(NW, NCH, CH)
    dst = edge_index[1].reshape(NW, NCH, CH)

    layer_input = _linear(x, W_lin, b_lin)

    sc_gather, sc_scatter = _sc_kernels()
    ls = sc_gather(layer_input, src).reshape(E, H)

    M2 = We2.reshape(H, H, H).transpose(0, 2, 1).reshape(H * H, H)
    B2T = be2.reshape(H, H).T
    eye = jnp.eye(H, dtype=f32)
    ones_row = jnp.ones((1, H), dtype=f32)
    Rk = jnp.kron(eye, ones_row)
    Rj = jnp.kron(ones_row, eye)
    msg = _edge_messages(edge_attr, ls, We1, be1, Rk, Rj, M2, B2T)

    zeros = jnp.zeros((STRIPE, H), f32)
    u2 = sc_scatter(msg.reshape(NW, NCH, CH, H), dst, zeros)

    gru_w = (
        W_ih[0:H].T, W_ih[H:2 * H].T, W_ih[2 * H:3 * H].T,
        W_hh[0:H].T, W_hh[H:2 * H].T, W_hh[2 * H:3 * H].T,
        (b_ih[0:H] + b_hh[0:H]).reshape(1, H),
        (b_ih[H:2 * H] + b_hh[H:2 * H]).reshape(1, H),
        b_ih[2 * H:3 * H].reshape(1, H),
        b_hh[2 * H:3 * H].reshape(1, H),
    )
    lstm_w = (
        Wl_ih[0:H].T, Wl_ih[H:2 * H].T, Wl_ih[2 * H:3 * H].T, Wl_ih[3 * H:4 * H].T,
        Wl_hh[0:H].T, Wl_hh[H:2 * H].T, Wl_hh[2 * H:3 * H].T, Wl_hh[3 * H:4 * H].T,
        (bl_ih[0:H] + bl_hh[0:H]).reshape(1, H),
        (bl_ih[H:2 * H] + bl_hh[H:2 * H]).reshape(1, H),
        (bl_ih[2 * H:3 * H] + bl_hh[2 * H:3 * H]).reshape(1, H),
        (bl_ih[3 * H:4 * H] + bl_hh[3 * H:4 * H]).reshape(1, H),
    )
    graph_feature, node_feature = _tail(
        u2, layer_input, node2graph.reshape(N, 1), gru_w, lstm_w)
    return graph_feature, node_feature


# speedup vs baseline: 5.7838x; 1.0214x over previous
import functools

import jax
import jax.numpy as jnp
from jax import lax
from jax.experimental import pallas as pl
from jax.experimental.pallas import tpu as pltpu
from jax.experimental.pallas import tpu_sc as plsc

N = 10000
E = 160000
D_IN = 128
H = 16
G = 64
S2S_STEPS = 3

NC = 2
NS = 16
NW = NC * NS
EPW = E // NW
CH = 100
NCH = EPW // CH
STRIPE = N // NS

@functools.cache
def _sc_kernels():
    mesh = plsc.VectorSubcoreMesh(core_axis_name="c", subcore_axis_name="s")

    @functools.partial(
        pl.kernel,
        mesh=mesh,
        out_type=jax.ShapeDtypeStruct((E, H), jnp.float32),
        scratch_types=[
            pltpu.VMEM((NCH, CH), jnp.int32),
            pltpu.VMEM((EPW, H), jnp.float32),
            pltpu.SemaphoreType.DMA,
        ],
        compiler_params=pltpu.CompilerParams(use_tc_tiling_on_sc=False),
    )
    def sc_gather(table_hbm, idx_hbm, out_hbm, idx_v, rows_v, sem):
        wid = lax.axis_index("s") * NC + lax.axis_index("c")
        base = wid * EPW
        pltpu.sync_copy(idx_hbm.at[wid], idx_v)

        def fire(j, carry):
            pltpu.async_copy(table_hbm.at[idx_v.at[j]],
                             rows_v.at[pl.ds(j * CH, CH)], sem)
            return carry

        lax.fori_loop(0, NCH, fire, 0)

        def drain(j, carry):
            pltpu.make_async_copy(table_hbm.at[idx_v.at[0]],
                                  rows_v.at[pl.ds(0, CH)], sem).wait()
            return carry

        lax.fori_loop(0, NCH, drain, 0)
        pltpu.sync_copy(rows_v, out_hbm.at[pl.ds(base, EPW)])

    @functools.partial(
        pl.kernel,
        mesh=mesh,
        out_type=jax.ShapeDtypeStruct((NC * N, H), jnp.float32),
        scratch_types=[
            pltpu.VMEM((NCH, CH), jnp.int32),
            pltpu.VMEM((EPW, H), jnp.float32),
            pltpu.VMEM((STRIPE, H), jnp.float32),
            pltpu.VMEM_SHARED((N, H), jnp.float32),
            pltpu.SemaphoreType.DMA,
        ],
        compiler_params=pltpu.CompilerParams(use_tc_tiling_on_sc=False),
    )
    def sc_scatter(msg_hbm, dst_hbm, zeros_hbm, out_hbm,
                   dst_v, msg_v, stripe_v, acc_sh, sem):
        cid = lax.axis_index("c")
        sid = lax.axis_index("s")
        wid = sid * NC + cid
        base = wid * EPW
        pltpu.sync_copy(zeros_hbm, stripe_v)
        pltpu.sync_copy(stripe_v, acc_sh.at[pl.ds(sid * STRIPE, STRIPE)])
        plsc.subcore_barrier()
        pltpu.sync_copy(dst_hbm.at[wid], dst_v)
        pltpu.sync_copy(msg_hbm.at[pl.ds(base, EPW)], msg_v)

        def fire(j, carry):
            pltpu.async_copy(msg_v.at[pl.ds(j * CH, CH)],
                             acc_sh.at[dst_v.at[j]], sem, add=True)
            return carry

        lax.fori_loop(0, NCH, fire, 0)

        def drain(j, carry):
            pltpu.make_async_copy(msg_v.at[pl.ds(0, CH)],
                                  acc_sh.at[dst_v.at[0]], sem).wait()
            return carry

        lax.fori_loop(0, NCH, drain, 0)
        plsc.subcore_barrier()
        pltpu.sync_copy(acc_sh.at[pl.ds(sid * STRIPE, STRIPE)], stripe_v)
        pltpu.sync_copy(stripe_v,
                        out_hbm.at[pl.ds(cid * N + sid * STRIPE, STRIPE)])

    return sc_gather, sc_scatter


def _lin_body(x_ref, w_ref, b_ref, o_ref):
    o_ref[...] = (
        jnp.dot(x_ref[...], w_ref[...], preferred_element_type=jnp.float32)
        + b_ref[...]
    )


def _edge_body(ea_ref, ls_ref, we1_ref, be1_ref, rk_ref, rj_ref, m2_ref,
               b2t_ref, o_ref):
    f32 = jnp.float32
    ea = ea_ref[...]
    ls = ls_ref[...]
    hmid = jnp.maximum(
        jnp.dot(ea, we1_ref[...], preferred_element_type=f32) + be1_ref[...],
        0.0,
    )
    hr = jnp.dot(hmid, rk_ref[...], preferred_element_type=f32)
    lt = jnp.dot(ls, rj_ref[...], preferred_element_type=f32)
    o_ref[...] = (
        jnp.dot(hr * lt, m2_ref[...], preferred_element_type=f32)
        + jnp.dot(ls, b2t_ref[...], preferred_element_type=f32)
    )


def _tail_body(u2_ref, li_ref, n2g_ref,
               wir_ref, wiz_ref, win_ref, whr_ref, whz_ref, whn_ref,
               br_ref, bz_ref, bin_ref, bhn_ref,
               wlii_ref, wlif_ref, wlig_ref, wlio_ref,
               wlhi_ref, wlhf_ref, wlhg_ref, wlho_ref,
               bli_ref, blf_ref, blg_ref, blo_ref,
               gf_ref, nf_ref):
    f32 = jnp.float32
    dot = lambda a, b: jnp.dot(a, b, preferred_element_type=f32)
    upd = u2_ref[0:N, :] + u2_ref[N:2 * N, :]
    mp = jnp.maximum(upd, 0.0)
    hx = li_ref[...]
    r = jax.nn.sigmoid(dot(mp, wir_ref[...]) + dot(hx, whr_ref[...]) + br_ref[...])
    z = jax.nn.sigmoid(dot(mp, wiz_ref[...]) + dot(hx, whz_ref[...]) + bz_ref[...])
    gn = jnp.tanh(dot(mp, win_ref[...]) + bin_ref[...]
                  + r * (dot(hx, whn_ref[...]) + bhn_ref[...]))
    hidden = (1.0 - z) * gn + z * hx
    nf_ref[...] = hidden
    n2g = n2g_ref[...]
    gids = lax.broadcasted_iota(jnp.int32, (N, G), 1)
    onehot_b = n2g == gids
    onehot_f = onehot_b.astype(f32)
    qstar = jnp.zeros((G, 2 * H), f32)
    h_l = jnp.zeros((G, H), f32)
    c_l = jnp.zeros((G, H), f32)
    for _ in range(S2S_STEPS):
        ig = jax.nn.sigmoid(dot(qstar, wlii_ref[...]) + dot(h_l, wlhi_ref[...]) + bli_ref[...])
        fg = jax.nn.sigmoid(dot(qstar, wlif_ref[...]) + dot(h_l, wlhf_ref[...]) + blf_ref[...])
        gg = jnp.tanh(dot(qstar, wlig_ref[...]) + dot(h_l, wlhg_ref[...]) + blg_ref[...])
        og = jax.nn.sigmoid(dot(qstar, wlio_ref[...]) + dot(h_l, wlho_ref[...]) + blo_ref[...])
        c_l = fg * c_l + ig * gg
        h_l = og * jnp.tanh(c_l)
        q = h_l
        qn = dot(onehot_f, q)
        e = jnp.sum(hidden * qn, axis=1, keepdims=True)
        em = jnp.where(onehot_b, e, -jnp.inf)
        m = jnp.max(em, axis=0, keepdims=True)
        m = jnp.where(m > -jnp.inf, m, 0.0)
        a = jnp.where(onehot_b, jnp.exp(e - m), 0.0)
        s = jnp.sum(a, axis=0, keepdims=True)
        w = a / (s + 1e-12)
        rr = lax.dot_general(w, hidden, (((0,), (0,)), ((), ())),
                             preferred_element_type=f32)
        qstar = jnp.concatenate([q, rr], axis=1)
    gf_ref[...] = qstar


def _linear(x, W, b):
    blk = 2000
    return pl.pallas_call(
        _lin_body,
        grid=(N // blk,),
        in_specs=[
            pl.BlockSpec((blk, D_IN), lambda i: (i, 0)),
            pl.BlockSpec((D_IN, H), lambda i: (0, 0)),
            pl.BlockSpec((1, H), lambda i: (0, 0)),
        ],
        out_specs=pl.BlockSpec((blk, H), lambda i: (i, 0)),
        out_shape=jax.ShapeDtypeStruct((N, H), jnp.float32),
    )(x, W, b.reshape(1, H))


def _edge_messages(edge_attr, ls, We1, be1, Rk, Rj, M2, B2T):
    blk = 4000
    full = lambda shape: pl.BlockSpec(shape, lambda i: tuple(0 for _ in shape))
    return pl.pallas_call(
        _edge_body,
        grid=(E // blk,),
        in_specs=[
            pl.BlockSpec((blk, H), lambda i: (i, 0)),
            pl.BlockSpec((blk, H), lambda i: (i, 0)),
            full((H, H)),
            full((1, H)),
            full((H, H * H)),
            full((H, H * H)),
            full((H * H, H)),
            full((H, H)),
        ],
        out_specs=pl.BlockSpec((blk, H), lambda i: (i, 0)),
        out_shape=jax.ShapeDtypeStruct((E, H), jnp.float32),
    )(edge_attr, ls, We1, be1.reshape(1, H), Rk, Rj, M2, B2T)


def _tail(u2, layer_input, n2g, gru_w, lstm_w):
    return pl.pallas_call(
        _tail_body,
        out_shape=(
            jax.ShapeDtypeStruct((G, 2 * H), jnp.float32),
            jax.ShapeDtypeStruct((N, H), jnp.float32),
        ),
    )(u2, layer_input, n2g, *gru_w, *lstm_w)


def kernel(x, edge_index, edge_attr, node2graph, W_lin, b_lin, We1, be1, We2,
           be2, W_ih, W_hh, b_ih, b_hh, Wl_ih, Wl_hh, bl_ih, bl_hh):
    f32 = jnp.float32
    src = edge_index[0].reshape(NW, NCH, CH)
    dst = edge_index[1].reshape(NW, NCH, CH)

    layer_input = _linear(x, W_lin, b_lin)

    sc_gather, sc_scatter = _sc_kernels()
    ls = sc_gather(layer_input, src)

    M2 = We2.reshape(H, H, H).transpose(0, 2, 1).reshape(H * H, H)
    B2T = be2.reshape(H, H).T
    eye = jnp.eye(H, dtype=f32)
    ones_row = jnp.ones((1, H), dtype=f32)
    Rk = jnp.kron(eye, ones_row)
    Rj = jnp.kron(ones_row, eye)
    msg = _edge_messages(edge_attr, ls, We1, be1, Rk, Rj, M2, B2T)

    zeros = jnp.zeros((STRIPE, H), f32)
    u2 = sc_scatter(msg, dst, zeros)

    gru_w = (
        W_ih[0:H].T, W_ih[H:2 * H].T, W_ih[2 * H:3 * H].T,
        W_hh[0:H].T, W_hh[H:2 * H].T, W_hh[2 * H:3 * H].T,
        (b_ih[0:H] + b_hh[0:H]).reshape(1, H),
        (b_ih[H:2 * H] + b_hh[H:2 * H]).reshape(1, H),
        b_ih[2 * H:3 * H].reshape(1, H),
        b_hh[2 * H:3 * H].reshape(1, H),
    )
    lstm_w = (
        Wl_ih[0:H].T, Wl_ih[H:2 * H].T, Wl_ih[2 * H:3 * H].T, Wl_ih[3 * H:4 * H].T,
        Wl_hh[0:H].T, Wl_hh[H:2 * H].T, Wl_hh[2 * H:3 * H].T, Wl_hh[3 * H:4 * H].T,
        (bl_ih[0:H] + bl_hh[0:H]).reshape(1, H),
        (bl_ih[H:2 * H] + bl_hh[H:2 * H]).reshape(1, H),
        (bl_ih[2 * H:3 * H] + bl_hh[2 * H:3 * H]).reshape(1, H),
        (bl_ih[3 * H:4 * H] + bl_hh[3 * H:4 * H]).reshape(1, H),
    )
    graph_feature, node_feature = _tail(
        u2, layer_input, node2graph.reshape(N, 1), gru_w, lstm_w)
    return graph_feature, node_feature


# speedup vs baseline: 7.9913x; 1.3817x over previous
import functools

import jax
import jax.numpy as jnp
from jax import lax
from jax.experimental import pallas as pl
from jax.experimental.pallas import tpu as pltpu
from jax.experimental.pallas import tpu_sc as plsc

N = 10000
E = 160000
D_IN = 128
H = 16
G = 64
S2S_STEPS = 3

NC = 2
NS = 16
NW = NC * NS
EPW = E // NW
CH = 100
NCH = EPW // CH
STRIPE = N // NS

@functools.cache
def _sc_kernels():
    mesh = plsc.VectorSubcoreMesh(core_axis_name="c", subcore_axis_name="s")

    @functools.partial(
        pl.kernel,
        mesh=mesh,
        out_type=jax.ShapeDtypeStruct((E, H), jnp.float32),
        scratch_types=[
            pltpu.VMEM((NCH, CH), jnp.int32),
            pltpu.VMEM((EPW, H), jnp.float32),
            pltpu.SemaphoreType.DMA,
        ],
        compiler_params=pltpu.CompilerParams(use_tc_tiling_on_sc=False),
    )
    def sc_gather(table_hbm, idx_hbm, out_hbm, idx_v, rows_v, sem):
        wid = lax.axis_index("s") * NC + lax.axis_index("c")
        base = wid * EPW
        pltpu.sync_copy(idx_hbm.at[wid], idx_v)

        def fire(j, carry):
            pltpu.async_copy(table_hbm.at[idx_v.at[j]],
                             rows_v.at[pl.ds(j * CH, CH)], sem)
            return carry

        lax.fori_loop(0, NCH, fire, 0)

        def drain(j, carry):
            pltpu.make_async_copy(table_hbm.at[idx_v.at[0]],
                                  rows_v.at[pl.ds(0, CH)], sem).wait()
            return carry

        lax.fori_loop(0, NCH, drain, 0)
        pltpu.sync_copy(rows_v, out_hbm.at[pl.ds(base, EPW)])

    @functools.partial(
        pl.kernel,
        mesh=mesh,
        out_type=jax.ShapeDtypeStruct((NC * N, H), jnp.float32),
        scratch_types=[
            pltpu.VMEM((NCH, CH), jnp.int32),
            pltpu.VMEM((EPW, H), jnp.float32),
            pltpu.VMEM((STRIPE, H), jnp.float32),
            pltpu.VMEM_SHARED((N, H), jnp.float32),
            pltpu.SemaphoreType.DMA,
        ],
        compiler_params=pltpu.CompilerParams(use_tc_tiling_on_sc=False),
    )
    def sc_scatter(msg_hbm, dst_hbm, zeros_hbm, out_hbm,
                   dst_v, msg_v, stripe_v, acc_sh, sem):
        cid = lax.axis_index("c")
        sid = lax.axis_index("s")
        wid = sid * NC + cid
        base = wid * EPW
        pltpu.sync_copy(zeros_hbm, stripe_v)
        pltpu.sync_copy(stripe_v, acc_sh.at[pl.ds(sid * STRIPE, STRIPE)])
        plsc.subcore_barrier()
        pltpu.sync_copy(dst_hbm.at[wid], dst_v)
        pltpu.sync_copy(msg_hbm.at[pl.ds(base, EPW)], msg_v)

        def fire(j, carry):
            pltpu.async_copy(msg_v.at[pl.ds(j * CH, CH)],
                             acc_sh.at[dst_v.at[j]], sem, add=True)
            return carry

        lax.fori_loop(0, NCH, fire, 0)

        def drain(j, carry):
            pltpu.make_async_copy(msg_v.at[pl.ds(0, CH)],
                                  acc_sh.at[dst_v.at[0]], sem).wait()
            return carry

        lax.fori_loop(0, NCH, drain, 0)
        plsc.subcore_barrier()
        pltpu.sync_copy(acc_sh.at[pl.ds(sid * STRIPE, STRIPE)], stripe_v)
        pltpu.sync_copy(stripe_v,
                        out_hbm.at[pl.ds(cid * N + sid * STRIPE, STRIPE)])

    return sc_gather, sc_scatter


def _lin_body(x_ref, w_ref, b_ref, o_ref):
    o_ref[...] = (
        jnp.dot(x_ref[...], w_ref[...], preferred_element_type=jnp.float32)
        + b_ref[...]
    )


def _edge_body(ea_ref, ls_ref, we1_ref, be1_ref, rk_ref, rj_ref, m2_ref,
               b2t_ref, o_ref):
    f32 = jnp.float32
    ea = ea_ref[...]
    ls = ls_ref[...]
    hmid = jnp.maximum(
        jnp.dot(ea, we1_ref[...], preferred_element_type=f32) + be1_ref[...],
        0.0,
    )
    hr = jnp.dot(hmid, rk_ref[...], preferred_element_type=f32)
    lt = jnp.dot(ls, rj_ref[...], preferred_element_type=f32)
    o_ref[...] = (
        jnp.dot(hr * lt, m2_ref[...], preferred_element_type=f32)
        + jnp.dot(ls, b2t_ref[...], preferred_element_type=f32)
    )


def _tail_body(u2_ref, li_ref, n2g_ref,
               wir_ref, wiz_ref, win_ref, whr_ref, whz_ref, whn_ref,
               br_ref, bz_ref, bin_ref, bhn_ref,
               wlii_ref, wlif_ref, wlig_ref, wlio_ref,
               wlhi_ref, wlhf_ref, wlhg_ref, wlho_ref,
               bli_ref, blf_ref, blg_ref, blo_ref,
               gf_ref, nf_ref):
    f32 = jnp.float32
    dot = lambda a, b: jnp.dot(a, b, preferred_element_type=f32)
    upd = u2_ref[0:N, :] + u2_ref[N:2 * N, :]
    mp = jnp.maximum(upd, 0.0)
    hx = li_ref[...]
    r = jax.nn.sigmoid(dot(mp, wir_ref[...]) + dot(hx, whr_ref[...]) + br_ref[...])
    z = jax.nn.sigmoid(dot(mp, wiz_ref[...]) + dot(hx, whz_ref[...]) + bz_ref[...])
    gn = jnp.tanh(dot(mp, win_ref[...]) + bin_ref[...]
                  + r * (dot(hx, whn_ref[...]) + bhn_ref[...]))
    hidden = (1.0 - z) * gn + z * hx
    nf_ref[...] = hidden
    n2g = n2g_ref[...]
    gids = lax.broadcasted_iota(jnp.int32, (N, G), 1)
    onehot_b = n2g == gids
    onehot_f = onehot_b.astype(f32)
    qstar = jnp.zeros((G, 2 * H), f32)
    h_l = jnp.zeros((G, H), f32)
    c_l = jnp.zeros((G, H), f32)
    for _ in range(S2S_STEPS):
        ig = jax.nn.sigmoid(dot(qstar, wlii_ref[...]) + dot(h_l, wlhi_ref[...]) + bli_ref[...])
        fg = jax.nn.sigmoid(dot(qstar, wlif_ref[...]) + dot(h_l, wlhf_ref[...]) + blf_ref[...])
        gg = jnp.tanh(dot(qstar, wlig_ref[...]) + dot(h_l, wlhg_ref[...]) + blg_ref[...])
        og = jax.nn.sigmoid(dot(qstar, wlio_ref[...]) + dot(h_l, wlho_ref[...]) + blo_ref[...])
        c_l = fg * c_l + ig * gg
        h_l = og * jnp.tanh(c_l)
        q = h_l
        qn = dot(onehot_f, q)
        e = jnp.sum(hidden * qn, axis=1, keepdims=True)
        em = jnp.where(onehot_b, e, -jnp.inf)
        m = jnp.max(em, axis=0, keepdims=True)
        m = jnp.where(m > -jnp.inf, m, 0.0)
        a = jnp.where(onehot_b, jnp.exp(e - m), 0.0)
        s = jnp.sum(a, axis=0, keepdims=True)
        w = a / (s + 1e-12)
        rr = lax.dot_general(w, hidden, (((0,), (0,)), ((), ())),
                             preferred_element_type=f32)
        qstar = jnp.concatenate([q, rr], axis=1)
    gf_ref[...] = qstar


_NP = N // 8
_EP = E // 8


def _linear(xp, Wp, bp):
    return pl.pallas_call(
        _lin_body,
        out_shape=jax.ShapeDtypeStruct((_NP, 8 * H), jnp.float32),
    )(xp, Wp, bp)


def _edge_messages(eap, lsp, We1p, be1p, Rkp, Rjp, M2p, B2Tp):
    pk = 1000
    full = lambda shape: pl.BlockSpec(shape, lambda i: tuple(0 for _ in shape))
    return pl.pallas_call(
        _edge_body,
        grid=(_EP // pk,),
        in_specs=[
            pl.BlockSpec((pk, 8 * H), lambda i: (i, 0)),
            pl.BlockSpec((pk, 8 * H), lambda i: (i, 0)),
            full((8 * H, 8 * H)),
            full((1, 8 * H)),
            full((8 * H, 8 * H * H)),
            full((8 * H, 8 * H * H)),
            full((8 * H * H, 8 * H)),
            full((8 * H, 8 * H)),
        ],
        out_specs=pl.BlockSpec((pk, 8 * H), lambda i: (i, 0)),
        out_shape=jax.ShapeDtypeStruct((_EP, 8 * H), jnp.float32),
    )(eap, lsp, We1p, be1p, Rkp, Rjp, M2p, B2Tp)


def _tail(u2, layer_input, n2g, gru_w, lstm_w):
    return pl.pallas_call(
        _tail_body,
        out_shape=(
            jax.ShapeDtypeStruct((G, 2 * H), jnp.float32),
            jax.ShapeDtypeStruct((N, H), jnp.float32),
        ),
    )(u2, layer_input, n2g, *gru_w, *lstm_w)


def kernel(x, edge_index, edge_attr, node2graph, W_lin, b_lin, We1, be1, We2,
           be2, W_ih, W_hh, b_ih, b_hh, Wl_ih, Wl_hh, bl_ih, bl_hh):
    f32 = jnp.float32
    src = edge_index[0].reshape(NW, NCH, CH)
    dst = edge_index[1].reshape(NW, NCH, CH)

    I8 = jnp.eye(8, dtype=f32)
    layer_input = _linear(
        x.reshape(_NP, 8 * D_IN),
        jnp.kron(I8, W_lin),
        jnp.tile(b_lin, 8).reshape(1, 8 * H),
    ).reshape(N, H)

    sc_gather, sc_scatter = _sc_kernels()
    ls = sc_gather(layer_input, src)

    M2 = We2.reshape(H, H, H).transpose(0, 2, 1).reshape(H * H, H)
    B2T = be2.reshape(H, H).T
    eye = jnp.eye(H, dtype=f32)
    ones_row = jnp.ones((1, H), dtype=f32)
    Rk = jnp.kron(eye, ones_row)
    Rj = jnp.kron(ones_row, eye)
    msg = _edge_messages(
        edge_attr.reshape(_EP, 8 * H),
        ls.reshape(_EP, 8 * H),
        jnp.kron(I8, We1),
        jnp.tile(be1, 8).reshape(1, 8 * H),
        jnp.kron(I8, Rk),
        jnp.kron(I8, Rj),
        jnp.kron(I8, M2),
        jnp.kron(I8, B2T),
    ).reshape(E, H)

    zeros = jnp.zeros((STRIPE, H), f32)
    u2 = sc_scatter(msg, dst, zeros)

    gru_w = (
        W_ih[0:H].T, W_ih[H:2 * H].T, W_ih[2 * H:3 * H].T,
        W_hh[0:H].T, W_hh[H:2 * H].T, W_hh[2 * H:3 * H].T,
        (b_ih[0:H] + b_hh[0:H]).reshape(1, H),
        (b_ih[H:2 * H] + b_hh[H:2 * H]).reshape(1, H),
        b_ih[2 * H:3 * H].reshape(1, H),
        b_hh[2 * H:3 * H].reshape(1, H),
    )
    lstm_w = (
        Wl_ih[0:H].T, Wl_ih[H:2 * H].T, Wl_ih[2 * H:3 * H].T, Wl_ih[3 * H:4 * H].T,
        Wl_hh[0:H].T, Wl_hh[H:2 * H].T, Wl_hh[2 * H:3 * H].T, Wl_hh[3 * H:4 * H].T,
        (bl_ih[0:H] + bl_hh[0:H]).reshape(1, H),
        (bl_ih[H:2 * H] + bl_hh[H:2 * H]).reshape(1, H),
        (bl_ih[2 * H:3 * H] + bl_hh[2 * H:3 * H]).reshape(1, H),
        (bl_ih[3 * H:4 * H] + bl_hh[3 * H:4 * H]).reshape(1, H),
    )
    graph_feature, node_feature = _tail(
        u2, layer_input, node2graph.reshape(N, 1), gru_w, lstm_w)
    return graph_feature, node_feature


# speedup vs baseline: 8.1684x; 1.0222x over previous
import functools

import jax
import jax.numpy as jnp
from jax import lax
from jax.experimental import pallas as pl
from jax.experimental.pallas import tpu as pltpu
from jax.experimental.pallas import tpu_sc as plsc

N = 10000
E = 160000
D_IN = 128
H = 16
G = 64
S2S_STEPS = 3

NC = 2
NS = 16
NW = NC * NS
EPW = E // NW
CH = 100
NCH = EPW // CH
STRIPE = N // NS

@functools.cache
def _sc_kernels():
    mesh = plsc.VectorSubcoreMesh(core_axis_name="c", subcore_axis_name="s")

    @functools.partial(
        pl.kernel,
        mesh=mesh,
        out_type=jax.ShapeDtypeStruct((E, H), jnp.float32),
        scratch_types=[
            pltpu.VMEM((NCH, CH), jnp.int32),
            pltpu.VMEM((EPW, H), jnp.float32),
            pltpu.SemaphoreType.DMA,
        ],
        compiler_params=pltpu.CompilerParams(use_tc_tiling_on_sc=False),
    )
    def sc_gather(table_hbm, idx_hbm, out_hbm, idx_v, rows_v, sem):
        wid = lax.axis_index("s") * NC + lax.axis_index("c")
        base = wid * EPW
        pltpu.sync_copy(idx_hbm.at[wid], idx_v)

        def fire(j, carry):
            pltpu.async_copy(table_hbm.at[idx_v.at[j]],
                             rows_v.at[pl.ds(j * CH, CH)], sem)
            return carry

        lax.fori_loop(0, NCH, fire, 0)

        def drain(j, carry):
            pltpu.make_async_copy(table_hbm.at[idx_v.at[0]],
                                  rows_v.at[pl.ds(0, CH)], sem).wait()
            return carry

        lax.fori_loop(0, NCH, drain, 0)
        pltpu.sync_copy(rows_v, out_hbm.at[pl.ds(base, EPW)])

    @functools.partial(
        pl.kernel,
        mesh=mesh,
        out_type=jax.ShapeDtypeStruct((NC * N, H), jnp.float32),
        scratch_types=[
            pltpu.VMEM((NCH, CH), jnp.int32),
            pltpu.VMEM((EPW, H), jnp.float32),
            pltpu.VMEM((STRIPE, H), jnp.float32),
            pltpu.VMEM_SHARED((N, H), jnp.float32),
            pltpu.SemaphoreType.DMA,
        ],
        compiler_params=pltpu.CompilerParams(use_tc_tiling_on_sc=False),
    )
    def sc_scatter(msg_hbm, dst_hbm, zeros_hbm, out_hbm,
                   dst_v, msg_v, stripe_v, acc_sh, sem):
        cid = lax.axis_index("c")
        sid = lax.axis_index("s")
        wid = sid * NC + cid
        base = wid * EPW
        pltpu.sync_copy(zeros_hbm, stripe_v)
        pltpu.sync_copy(stripe_v, acc_sh.at[pl.ds(sid * STRIPE, STRIPE)])
        plsc.subcore_barrier()
        pltpu.sync_copy(dst_hbm.at[wid], dst_v)
        pltpu.sync_copy(msg_hbm.at[pl.ds(base, EPW)], msg_v)

        def fire(j, carry):
            pltpu.async_copy(msg_v.at[pl.ds(j * CH, CH)],
                             acc_sh.at[dst_v.at[j]], sem, add=True)
            return carry

        lax.fori_loop(0, NCH, fire, 0)

        def drain(j, carry):
            pltpu.make_async_copy(msg_v.at[pl.ds(0, CH)],
                                  acc_sh.at[dst_v.at[0]], sem).wait()
            return carry

        lax.fori_loop(0, NCH, drain, 0)
        plsc.subcore_barrier()
        pltpu.sync_copy(acc_sh.at[pl.ds(sid * STRIPE, STRIPE)], stripe_v)
        pltpu.sync_copy(stripe_v,
                        out_hbm.at[pl.ds(cid * N + sid * STRIPE, STRIPE)])

    return sc_gather, sc_scatter


def _lin_body(x_ref, w_ref, b_ref, o_ref):
    o_ref[...] = (
        jnp.dot(x_ref[...], w_ref[...], preferred_element_type=jnp.float32)
        + b_ref[...]
    )


def _edge_body(ea_ref, ls_ref, we1_ref, be1_ref, rk_ref, rj_ref, m2_ref,
               b2t_ref, o_ref):
    f32 = jnp.float32
    bf = jnp.bfloat16
    ea = ea_ref[...]
    ls = ls_ref[...]
    hmid = jnp.maximum(
        jnp.dot(ea, we1_ref[...], preferred_element_type=f32) + be1_ref[...],
        0.0,
    )
    hr = jnp.dot(hmid.astype(bf), rk_ref[...], preferred_element_type=f32)
    lt = jnp.dot(ls.astype(bf), rj_ref[...], preferred_element_type=f32)
    o_ref[...] = (
        jnp.dot((hr * lt).astype(bf), m2_ref[...], preferred_element_type=f32)
        + jnp.dot(ls, b2t_ref[...], preferred_element_type=f32)
    )


def _tail_body(u2_ref, li_ref, n2g_ref,
               wir_ref, wiz_ref, win_ref, whr_ref, whz_ref, whn_ref,
               br_ref, bz_ref, bin_ref, bhn_ref,
               wlii_ref, wlif_ref, wlig_ref, wlio_ref,
               wlhi_ref, wlhf_ref, wlhg_ref, wlho_ref,
               bli_ref, blf_ref, blg_ref, blo_ref,
               gf_ref, nf_ref):
    f32 = jnp.float32
    dot = lambda a, b: jnp.dot(a, b, preferred_element_type=f32)
    upd = u2_ref[0:N, :] + u2_ref[N:2 * N, :]
    mp = jnp.maximum(upd, 0.0)
    hx = li_ref[...]
    r = jax.nn.sigmoid(dot(mp, wir_ref[...]) + dot(hx, whr_ref[...]) + br_ref[...])
    z = jax.nn.sigmoid(dot(mp, wiz_ref[...]) + dot(hx, whz_ref[...]) + bz_ref[...])
    gn = jnp.tanh(dot(mp, win_ref[...]) + bin_ref[...]
                  + r * (dot(hx, whn_ref[...]) + bhn_ref[...]))
    hidden = (1.0 - z) * gn + z * hx
    nf_ref[...] = hidden
    n2g = n2g_ref[...]
    gids = lax.broadcasted_iota(jnp.int32, (N, G), 1)
    onehot_b = n2g == gids
    onehot_f = onehot_b.astype(f32)
    qstar = jnp.zeros((G, 2 * H), f32)
    h_l = jnp.zeros((G, H), f32)
    c_l = jnp.zeros((G, H), f32)
    for _ in range(S2S_STEPS):
        ig = jax.nn.sigmoid(dot(qstar, wlii_ref[...]) + dot(h_l, wlhi_ref[...]) + bli_ref[...])
        fg = jax.nn.sigmoid(dot(qstar, wlif_ref[...]) + dot(h_l, wlhf_ref[...]) + blf_ref[...])
        gg = jnp.tanh(dot(qstar, wlig_ref[...]) + dot(h_l, wlhg_ref[...]) + blg_ref[...])
        og = jax.nn.sigmoid(dot(qstar, wlio_ref[...]) + dot(h_l, wlho_ref[...]) + blo_ref[...])
        c_l = fg * c_l + ig * gg
        h_l = og * jnp.tanh(c_l)
        q = h_l
        qn = dot(onehot_f, q)
        e = jnp.sum(hidden * qn, axis=1, keepdims=True)
        em = jnp.where(onehot_b, e, -jnp.inf)
        m = jnp.max(em, axis=0, keepdims=True)
        m = jnp.where(m > -jnp.inf, m, 0.0)
        a = jnp.where(onehot_b, jnp.exp(e - m), 0.0)
        s = jnp.sum(a, axis=0, keepdims=True)
        w = a / (s + 1e-12)
        rr = lax.dot_general(w, hidden, (((0,), (0,)), ((), ())),
                             preferred_element_type=f32)
        qstar = jnp.concatenate([q, rr], axis=1)
    gf_ref[...] = qstar


_NP = N // 8
_EP = E // 8


def _linear(xp, Wp, bp):
    return pl.pallas_call(
        _lin_body,
        out_shape=jax.ShapeDtypeStruct((_NP, 8 * H), jnp.float32),
    )(xp, Wp, bp)


def _edge_messages(eap, lsp, We1p, be1p, Rkp, Rjp, M2p, B2Tp):
    pk = 1000
    full = lambda shape: pl.BlockSpec(shape, lambda i: tuple(0 for _ in shape))
    return pl.pallas_call(
        _edge_body,
        grid=(_EP // pk,),
        in_specs=[
            pl.BlockSpec((pk, 8 * H), lambda i: (i, 0)),
            pl.BlockSpec((pk, 8 * H), lambda i: (i, 0)),
            full((8 * H, 8 * H)),
            full((1, 8 * H)),
            full((8 * H, 8 * H * H)),
            full((8 * H, 8 * H * H)),
            full((8 * H * H, 8 * H)),
            full((8 * H, 8 * H)),
        ],
        out_specs=pl.BlockSpec((pk, 8 * H), lambda i: (i, 0)),
        out_shape=jax.ShapeDtypeStruct((_EP, 8 * H), jnp.float32),
    )(eap, lsp, We1p, be1p, Rkp, Rjp, M2p, B2Tp)


def _tail(u2, layer_input, n2g, gru_w, lstm_w):
    return pl.pallas_call(
        _tail_body,
        out_shape=(
            jax.ShapeDtypeStruct((G, 2 * H), jnp.float32),
            jax.ShapeDtypeStruct((N, H), jnp.float32),
        ),
    )(u2, layer_input, n2g, *gru_w, *lstm_w)


def kernel(x, edge_index, edge_attr, node2graph, W_lin, b_lin, We1, be1, We2,
           be2, W_ih, W_hh, b_ih, b_hh, Wl_ih, Wl_hh, bl_ih, bl_hh):
    f32 = jnp.float32
    src = edge_index[0].reshape(NW, NCH, CH)
    dst = edge_index[1].reshape(NW, NCH, CH)

    I8 = jnp.eye(8, dtype=f32)
    layer_input = _linear(
        x.reshape(_NP, 8 * D_IN),
        jnp.kron(I8, W_lin),
        jnp.tile(b_lin, 8).reshape(1, 8 * H),
    ).reshape(N, H)

    sc_gather, sc_scatter = _sc_kernels()
    ls = sc_gather(layer_input, src)

    M2 = We2.reshape(H, H, H).transpose(0, 2, 1).reshape(H * H, H)
    B2T = be2.reshape(H, H).T
    eye = jnp.eye(H, dtype=f32)
    ones_row = jnp.ones((1, H), dtype=f32)
    Rk = jnp.kron(eye, ones_row)
    Rj = jnp.kron(ones_row, eye)
    bf = jnp.bfloat16
    msg = _edge_messages(
        edge_attr.reshape(_EP, 8 * H),
        ls.reshape(_EP, 8 * H),
        jnp.kron(I8, We1),
        jnp.tile(be1, 8).reshape(1, 8 * H),
        jnp.kron(I8, Rk).astype(bf),
        jnp.kron(I8, Rj).astype(bf),
        jnp.kron(I8, M2).astype(bf),
        jnp.kron(I8, B2T),
    ).reshape(E, H)

    zeros = jnp.zeros((STRIPE, H), f32)
    u2 = sc_scatter(msg, dst, zeros)

    gru_w = (
        W_ih[0:H].T, W_ih[H:2 * H].T, W_ih[2 * H:3 * H].T,
        W_hh[0:H].T, W_hh[H:2 * H].T, W_hh[2 * H:3 * H].T,
        (b_ih[0:H] + b_hh[0:H]).reshape(1, H),
        (b_ih[H:2 * H] + b_hh[H:2 * H]).reshape(1, H),
        b_ih[2 * H:3 * H].reshape(1, H),
        b_hh[2 * H:3 * H].reshape(1, H),
    )
    lstm_w = (
        Wl_ih[0:H].T, Wl_ih[H:2 * H].T, Wl_ih[2 * H:3 * H].T, Wl_ih[3 * H:4 * H].T,
        Wl_hh[0:H].T, Wl_hh[H:2 * H].T, Wl_hh[2 * H:3 * H].T, Wl_hh[3 * H:4 * H].T,
        (bl_ih[0:H] + bl_hh[0:H]).reshape(1, H),
        (bl_ih[H:2 * H] + bl_hh[H:2 * H]).reshape(1, H),
        (bl_ih[2 * H:3 * H] + bl_hh[2 * H:3 * H]).reshape(1, H),
        (bl_ih[3 * H:4 * H] + bl_hh[3 * H:4 * H]).reshape(1, H),
    )
    graph_feature, node_feature = _tail(
        u2, layer_input, node2graph.reshape(N, 1), gru_w, lstm_w)
    return graph_feature, node_feature


# speedup vs baseline: 8.2354x; 1.0082x over previous
import functools

import jax
import jax.numpy as jnp
from jax import lax
from jax.experimental import pallas as pl
from jax.experimental.pallas import tpu as pltpu
from jax.experimental.pallas import tpu_sc as plsc

N = 10000
E = 160000
D_IN = 128
H = 16
G = 64
S2S_STEPS = 3

NC = 2
NS = 16
NW = NC * NS
EPW = E // NW
CH = 100
NCH = EPW // CH
STRIPE = N // NS

@functools.cache
def _sc_kernels():
    mesh = plsc.VectorSubcoreMesh(core_axis_name="c", subcore_axis_name="s")

    @functools.partial(
        pl.kernel,
        mesh=mesh,
        out_type=jax.ShapeDtypeStruct((E, H), jnp.float32),
        scratch_types=[
            pltpu.VMEM((NCH, CH), jnp.int32),
            pltpu.VMEM((EPW, H), jnp.float32),
            pltpu.SemaphoreType.DMA,
        ],
        compiler_params=pltpu.CompilerParams(use_tc_tiling_on_sc=False),
    )
    def sc_gather(table_hbm, idx_hbm, out_hbm, idx_v, rows_v, sem):
        wid = lax.axis_index("s") * NC + lax.axis_index("c")
        base = wid * EPW
        pltpu.sync_copy(idx_hbm.at[wid], idx_v)

        def fire(j, carry):
            pltpu.async_copy(table_hbm.at[idx_v.at[j]],
                             rows_v.at[pl.ds(j * CH, CH)], sem)
            return carry

        lax.fori_loop(0, NCH, fire, 0)

        def drain(j, carry):
            pltpu.make_async_copy(table_hbm.at[idx_v.at[0]],
                                  rows_v.at[pl.ds(0, CH)], sem).wait()
            return carry

        lax.fori_loop(0, NCH, drain, 0)
        pltpu.sync_copy(rows_v, out_hbm.at[pl.ds(base, EPW)])

    @functools.partial(
        pl.kernel,
        mesh=mesh,
        out_type=jax.ShapeDtypeStruct((NC * N, H), jnp.float32),
        scratch_types=[
            pltpu.VMEM((NCH, CH), jnp.int32),
            pltpu.VMEM((EPW, H), jnp.float32),
            pltpu.VMEM((STRIPE, H), jnp.float32),
            pltpu.VMEM_SHARED((N, H), jnp.float32),
            pltpu.SemaphoreType.DMA,
        ],
        compiler_params=pltpu.CompilerParams(use_tc_tiling_on_sc=False),
    )
    def sc_scatter(msg_hbm, dst_hbm, zeros_hbm, out_hbm,
                   dst_v, msg_v, stripe_v, acc_sh, sem):
        cid = lax.axis_index("c")
        sid = lax.axis_index("s")
        wid = sid * NC + cid
        base = wid * EPW
        ld_d = pltpu.async_copy(dst_hbm.at[wid], dst_v, sem)
        ld_m = pltpu.async_copy(msg_hbm.at[pl.ds(base, EPW)], msg_v, sem)
        pltpu.sync_copy(zeros_hbm, stripe_v)
        pltpu.sync_copy(stripe_v, acc_sh.at[pl.ds(sid * STRIPE, STRIPE)])
        plsc.subcore_barrier()
        ld_d.wait()
        ld_m.wait()

        def fire(j, carry):
            pltpu.async_copy(msg_v.at[pl.ds(j * CH, CH)],
                             acc_sh.at[dst_v.at[j]], sem, add=True)
            return carry

        lax.fori_loop(0, NCH, fire, 0)

        def drain(j, carry):
            pltpu.make_async_copy(msg_v.at[pl.ds(0, CH)],
                                  acc_sh.at[dst_v.at[0]], sem).wait()
            return carry

        lax.fori_loop(0, NCH, drain, 0)
        plsc.subcore_barrier()
        pltpu.sync_copy(acc_sh.at[pl.ds(sid * STRIPE, STRIPE)], stripe_v)
        pltpu.sync_copy(stripe_v,
                        out_hbm.at[pl.ds(cid * N + sid * STRIPE, STRIPE)])

    return sc_gather, sc_scatter


def _lin_body(x_ref, w_ref, b_ref, o_ref):
    o_ref[...] = (
        jnp.dot(x_ref[...], w_ref[...], preferred_element_type=jnp.float32)
        + b_ref[...]
    )


def _edge_body(ea_ref, ls_ref, we1_ref, be1_ref, rk_ref, rj_ref, m2_ref,
               b2t_ref, o_ref):
    f32 = jnp.float32
    bf = jnp.bfloat16
    ea = ea_ref[...]
    ls = ls_ref[...]
    hmid = jnp.maximum(
        jnp.dot(ea, we1_ref[...], preferred_element_type=f32) + be1_ref[...],
        0.0,
    )
    hr = jnp.dot(hmid.astype(bf), rk_ref[...], preferred_element_type=f32)
    lt = jnp.dot(ls.astype(bf), rj_ref[...], preferred_element_type=f32)
    o_ref[...] = (
        jnp.dot((hr * lt).astype(bf), m2_ref[...], preferred_element_type=f32)
        + jnp.dot(ls, b2t_ref[...], preferred_element_type=f32)
    )


def _tail_body(u2_ref, li_ref, n2g_ref,
               wir_ref, wiz_ref, win_ref, whr_ref, whz_ref, whn_ref,
               br_ref, bz_ref, bin_ref, bhn_ref,
               wlii_ref, wlif_ref, wlig_ref, wlio_ref,
               wlhi_ref, wlhf_ref, wlhg_ref, wlho_ref,
               bli_ref, blf_ref, blg_ref, blo_ref,
               gf_ref, nf_ref):
    f32 = jnp.float32
    dot = lambda a, b: jnp.dot(a, b, preferred_element_type=f32)
    upd = u2_ref[0:N, :] + u2_ref[N:2 * N, :]
    mp = jnp.maximum(upd, 0.0)
    hx = li_ref[...]
    r = jax.nn.sigmoid(dot(mp, wir_ref[...]) + dot(hx, whr_ref[...]) + br_ref[...])
    z = jax.nn.sigmoid(dot(mp, wiz_ref[...]) + dot(hx, whz_ref[...]) + bz_ref[...])
    gn = jnp.tanh(dot(mp, win_ref[...]) + bin_ref[...]
                  + r * (dot(hx, whn_ref[...]) + bhn_ref[...]))
    hidden = (1.0 - z) * gn + z * hx
    nf_ref[...] = hidden
    n2g = n2g_ref[...]
    gids = lax.broadcasted_iota(jnp.int32, (N, G), 1)
    onehot_b = n2g == gids
    onehot_f = onehot_b.astype(f32)
    qstar = jnp.zeros((G, 2 * H), f32)
    h_l = jnp.zeros((G, H), f32)
    c_l = jnp.zeros((G, H), f32)
    for _ in range(S2S_STEPS):
        ig = jax.nn.sigmoid(dot(qstar, wlii_ref[...]) + dot(h_l, wlhi_ref[...]) + bli_ref[...])
        fg = jax.nn.sigmoid(dot(qstar, wlif_ref[...]) + dot(h_l, wlhf_ref[...]) + blf_ref[...])
        gg = jnp.tanh(dot(qstar, wlig_ref[...]) + dot(h_l, wlhg_ref[...]) + blg_ref[...])
        og = jax.nn.sigmoid(dot(qstar, wlio_ref[...]) + dot(h_l, wlho_ref[...]) + blo_ref[...])
        c_l = fg * c_l + ig * gg
        h_l = og * jnp.tanh(c_l)
        q = h_l
        qn = dot(onehot_f, q)
        e = jnp.sum(hidden * qn, axis=1, keepdims=True)
        em = jnp.where(onehot_b, e, -jnp.inf)
        m = jnp.max(em, axis=0, keepdims=True)
        m = jnp.where(m > -jnp.inf, m, 0.0)
        a = jnp.where(onehot_b, jnp.exp(e - m), 0.0)
        s = jnp.sum(a, axis=0, keepdims=True)
        w = a / (s + 1e-12)
        rr = lax.dot_general(w, hidden, (((0,), (0,)), ((), ())),
                             preferred_element_type=f32)
        qstar = jnp.concatenate([q, rr], axis=1)
    gf_ref[...] = qstar


_NP = N // 8
_EP = E // 8


def _linear(xp, Wp, bp):
    return pl.pallas_call(
        _lin_body,
        out_shape=jax.ShapeDtypeStruct((_NP, 8 * H), jnp.float32),
    )(xp, Wp, bp)


def _edge_messages(eap, lsp, We1p, be1p, Rkp, Rjp, M2p, B2Tp):
    pk = 1000
    full = lambda shape: pl.BlockSpec(shape, lambda i: tuple(0 for _ in shape))
    return pl.pallas_call(
        _edge_body,
        grid=(_EP // pk,),
        in_specs=[
            pl.BlockSpec((pk, 8 * H), lambda i: (i, 0)),
            pl.BlockSpec((pk, 8 * H), lambda i: (i, 0)),
            full((8 * H, 8 * H)),
            full((1, 8 * H)),
            full((8 * H, 8 * H * H)),
            full((8 * H, 8 * H * H)),
            full((8 * H * H, 8 * H)),
            full((8 * H, 8 * H)),
        ],
        out_specs=pl.BlockSpec((pk, 8 * H), lambda i: (i, 0)),
        out_shape=jax.ShapeDtypeStruct((_EP, 8 * H), jnp.float32),
    )(eap, lsp, We1p, be1p, Rkp, Rjp, M2p, B2Tp)


def _tail(u2, layer_input, n2g, gru_w, lstm_w):
    return pl.pallas_call(
        _tail_body,
        out_shape=(
            jax.ShapeDtypeStruct((G, 2 * H), jnp.float32),
            jax.ShapeDtypeStruct((N, H), jnp.float32),
        ),
    )(u2, layer_input, n2g, *gru_w, *lstm_w)


def kernel(x, edge_index, edge_attr, node2graph, W_lin, b_lin, We1, be1, We2,
           be2, W_ih, W_hh, b_ih, b_hh, Wl_ih, Wl_hh, bl_ih, bl_hh):
    f32 = jnp.float32
    src = edge_index[0].reshape(NW, NCH, CH)
    dst = edge_index[1].reshape(NW, NCH, CH)

    I8 = jnp.eye(8, dtype=f32)
    layer_input = _linear(
        x.reshape(_NP, 8 * D_IN),
        jnp.kron(I8, W_lin),
        jnp.tile(b_lin, 8).reshape(1, 8 * H),
    ).reshape(N, H)

    sc_gather, sc_scatter = _sc_kernels()
    ls = sc_gather(layer_input, src)

    M2 = We2.reshape(H, H, H).transpose(0, 2, 1).reshape(H * H, H)
    B2T = be2.reshape(H, H).T
    eye = jnp.eye(H, dtype=f32)
    ones_row = jnp.ones((1, H), dtype=f32)
    Rk = jnp.kron(eye, ones_row)
    Rj = jnp.kron(ones_row, eye)
    bf = jnp.bfloat16
    msg = _edge_messages(
        edge_attr.reshape(_EP, 8 * H),
        ls.reshape(_EP, 8 * H),
        jnp.kron(I8, We1),
        jnp.tile(be1, 8).reshape(1, 8 * H),
        jnp.kron(I8, Rk).astype(bf),
        jnp.kron(I8, Rj).astype(bf),
        jnp.kron(I8, M2).astype(bf),
        jnp.kron(I8, B2T),
    ).reshape(E, H)

    zeros = jnp.zeros((STRIPE, H), f32)
    u2 = sc_scatter(msg, dst, zeros)

    gru_w = (
        W_ih[0:H].T, W_ih[H:2 * H].T, W_ih[2 * H:3 * H].T,
        W_hh[0:H].T, W_hh[H:2 * H].T, W_hh[2 * H:3 * H].T,
        (b_ih[0:H] + b_hh[0:H]).reshape(1, H),
        (b_ih[H:2 * H] + b_hh[H:2 * H]).reshape(1, H),
        b_ih[2 * H:3 * H].reshape(1, H),
        b_hh[2 * H:3 * H].reshape(1, H),
    )
    lstm_w = (
        Wl_ih[0:H].T, Wl_ih[H:2 * H].T, Wl_ih[2 * H:3 * H].T, Wl_ih[3 * H:4 * H].T,
        Wl_hh[0:H].T, Wl_hh[H:2 * H].T, Wl_hh[2 * H:3 * H].T, Wl_hh[3 * H:4 * H].T,
        (bl_ih[0:H] + bl_hh[0:H]).reshape(1, H),
        (bl_ih[H:2 * H] + bl_hh[H:2 * H]).reshape(1, H),
        (bl_ih[2 * H:3 * H] + bl_hh[2 * H:3 * H]).reshape(1, H),
        (bl_ih[3 * H:4 * H] + bl_hh[3 * H:4 * H]).reshape(1, H),
    )
    graph_feature, node_feature = _tail(
        u2, layer_input, node2graph.reshape(N, 1), gru_w, lstm_w)
    return graph_feature, node_feature
